# baseline (speedup 1.0000x reference)
import jax
import jax.numpy as jnp
from jax import lax
from jax.experimental import pallas as pl
from jax.experimental.pallas import tpu as pltpu

_B, _P, _D, _H = 16, 2048, 1024, 24
_D4 = _D // 4
_CHUNK = 256
_NCHUNK = _P // _CHUNK
_XDIM = _D + _H * _D4
_KCHUNK = 1024
_NK = _XDIM // _KCHUNK


def _pool_body(s_ref, ctx_ref, wp_ref, bp_ref, out_ref, acc_ref):
    b = pl.program_id(0)
    c = pl.program_id(1)
    n = s_ref[b]
    bin_size = jnp.maximum(n // _H, 1)
    h = lax.broadcasted_iota(jnp.int32, (_H, 1), 0)
    start = h * bin_size
    end = jnp.minimum(start + bin_size, n)
    needed = jnp.minimum(_H * bin_size, n)

    @pl.when(c == 0)
    def _():
        acc_ref[...] = jnp.zeros_like(acc_ref)

    @pl.when(c * _CHUNK < needed)
    def _():
        pos = lax.broadcasted_iota(jnp.int32, (_H, _CHUNK), 1) + c * _CHUNK
        m = ((pos >= start) & (pos < end)).astype(jnp.float32)
        inv = 1.0 / jnp.maximum((end - start).astype(jnp.float32), 1.0)
        w = m * inv
        acc_ref[...] += jnp.dot(w, ctx_ref[0], preferred_element_type=jnp.float32)

    @pl.when(c == _NCHUNK - 1)
    def _():
        hourly = lax.dot_general(
            acc_ref[...], wp_ref[...], (((1,), (1,)), ((), ())),
            preferred_element_type=jnp.float32)
        hourly = hourly + bp_ref[...]
        nonempty = (start < n).astype(jnp.float32)
        out_ref[0] = hourly * nonempty


def _ctx_index(b, c, s):
    n = s[b]
    bin_size = jnp.maximum(n // _H, 1)
    needed = jnp.minimum(_H * bin_size, n)
    last = jnp.maximum(pl.cdiv(needed, _CHUNK) - 1, 0)
    return (b, jnp.minimum(c, last), 0)


def _mlp_body(x_ref, w1_ref, b1_ref, w2_ref, b2_ref, out_ref, acc_ref):
    k = pl.program_id(0)

    @pl.when(k == 0)
    def _():
        acc_ref[...] = jnp.zeros_like(acc_ref)

    acc_ref[...] += lax.dot_general(
        x_ref[...], w1_ref[...], (((1,), (1,)), ((), ())),
        preferred_element_type=jnp.float32)

    @pl.when(k == _NK - 1)
    def _():
        y = acc_ref[...] + b1_ref[...]
        y = 0.5 * y * (1.0 + lax.erf(y * 0.7071067811865476))
        out_ref[...] = lax.dot_general(
            y, w2_ref[...], (((1,), (1,)), ((), ())),
            preferred_element_type=jnp.float32) + b2_ref[...]


def kernel(day_embed, episode_ctx, n_episodes, Wp, bp, W1, b1, W2, b2):
    hourly = pl.pallas_call(
        _pool_body,
        grid_spec=pltpu.PrefetchScalarGridSpec(
            num_scalar_prefetch=1,
            grid=(_B, _NCHUNK),
            in_specs=[
                pl.BlockSpec((1, _CHUNK, _D), _ctx_index),
                pl.BlockSpec((_D4, _D), lambda b, c, s: (0, 0)),
                pl.BlockSpec((1, _D4), lambda b, c, s: (0, 0)),
            ],
            out_specs=pl.BlockSpec((1, _H, _D4), lambda b, c, s: (b, 0, 0)),
            scratch_shapes=[pltpu.VMEM((_H, _D), jnp.float32)],
        ),
        out_shape=jax.ShapeDtypeStruct((_B, _H, _D4), jnp.float32),
    )(n_episodes.astype(jnp.int32), episode_ctx, Wp, bp.reshape(1, _D4))

    x = jnp.concatenate([day_embed, hourly.reshape(_B, _H * _D4)], axis=-1)

    out = pl.pallas_call(
        _mlp_body,
        grid=(_NK,),
        in_specs=[
            pl.BlockSpec((_B, _KCHUNK), lambda k: (0, k)),
            pl.BlockSpec((_D, _KCHUNK), lambda k: (0, k)),
            pl.BlockSpec((1, _D), lambda k: (0, 0)),
            pl.BlockSpec((2, _D), lambda k: (0, 0)),
            pl.BlockSpec((1, 2), lambda k: (0, 0)),
        ],
        out_specs=pl.BlockSpec((_B, 2), lambda k: (0, 0)),
        out_shape=jax.ShapeDtypeStruct((_B, 2), jnp.float32),
        scratch_shapes=[pltpu.VMEM((_B, _D), jnp.float32)],
    )(x, W1, b1.reshape(1, _D), W2, b2.reshape(1, 2))

    return out


# speedup vs baseline: 1.3321x; 1.3321x over previous
import jax
import jax.numpy as jnp
from jax import lax
from jax.experimental import pallas as pl
from jax.experimental.pallas import tpu as pltpu

_B, _P, _D, _H = 16, 2048, 1024, 24
_D4 = _D // 4
_CHUNK = 256
_NBUF = 8
_XDIM = _D + _H * _D4
_KCHUNK = 1024
_NK = _XDIM // _KCHUNK


def _pool_body(s_ref, ctx_ref, wp_ref, bp_ref, out_ref, abuf, asem, acc_ref):
    def chunk_copy(b, i, slot):
        return pltpu.make_async_copy(
            ctx_ref.at[b, pl.ds(i * _CHUNK, _CHUNK), :], abuf.at[slot],
            asem.at[slot])

    h = lax.broadcasted_iota(jnp.int32, (_H, 1), 0)

    def per_sample(b, carry):
        n = s_ref[b]
        bin_size = jnp.maximum(n // _H, 1)
        start = h * bin_size
        end = jnp.minimum(start + bin_size, n)
        needed = jnp.minimum(_H * bin_size, n)
        nch = pl.cdiv(needed, _CHUNK)

        def prime(i, c):
            chunk_copy(b, i, lax.rem(i, _NBUF)).start()
            return c

        lax.fori_loop(0, jnp.minimum(nch, _NBUF), prime, 0)
        acc_ref[...] = jnp.zeros_like(acc_ref)

        def consume(i, c):
            slot = lax.rem(i, _NBUF)
            chunk_copy(b, i, slot).wait()
            pos = lax.broadcasted_iota(jnp.int32, (_H, _CHUNK), 1) + i * _CHUNK
            m = ((pos >= start) & (pos < end)).astype(jnp.bfloat16)
            acc_ref[...] += jnp.dot(m, abuf[slot].astype(jnp.bfloat16),
                                    preferred_element_type=jnp.float32)

            @pl.when(i + _NBUF < nch)
            def _():
                chunk_copy(b, i + _NBUF, slot).start()
            return c

        lax.fori_loop(0, nch, consume, 0)

        inv = 1.0 / jnp.maximum((end - start).astype(jnp.float32), 1.0)
        nonempty = (start < n).astype(jnp.float32)
        seg_mean = acc_ref[...] * (inv * nonempty)
        hourly = lax.dot_general(
            seg_mean, wp_ref[...], (((1,), (1,)), ((), ())),
            preferred_element_type=jnp.float32)
        out_ref[pl.ds(b, 1)] = (hourly + bp_ref[...] * nonempty)[None]
        return carry

    lax.fori_loop(0, _B, per_sample, 0)


def _mlp_body(x_ref, w1_ref, b1_ref, w2_ref, b2_ref, out_ref, acc_ref):
    k = pl.program_id(0)

    @pl.when(k == 0)
    def _():
        acc_ref[...] = jnp.zeros_like(acc_ref)

    acc_ref[...] += lax.dot_general(
        x_ref[...], w1_ref[...], (((1,), (1,)), ((), ())),
        preferred_element_type=jnp.float32)

    @pl.when(k == _NK - 1)
    def _():
        y = acc_ref[...] + b1_ref[...]
        y = 0.5 * y * (1.0 + lax.erf(y * 0.7071067811865476))
        out_ref[...] = lax.dot_general(
            y, w2_ref[...], (((1,), (1,)), ((), ())),
            preferred_element_type=jnp.float32) + b2_ref[...]


def kernel(day_embed, episode_ctx, n_episodes, Wp, bp, W1, b1, W2, b2):
    hourly = pl.pallas_call(
        _pool_body,
        grid_spec=pltpu.PrefetchScalarGridSpec(
            num_scalar_prefetch=1,
            grid=(1,),
            in_specs=[
                pl.BlockSpec(memory_space=pl.ANY),
                pl.BlockSpec((_D4, _D), lambda i, s: (0, 0)),
                pl.BlockSpec((1, _D4), lambda i, s: (0, 0)),
            ],
            out_specs=pl.BlockSpec((_B, _H, _D4), lambda i, s: (0, 0, 0)),
            scratch_shapes=[
                pltpu.VMEM((_NBUF, _CHUNK, _D), jnp.float32),
                pltpu.SemaphoreType.DMA((_NBUF,)),
                pltpu.VMEM((_H, _D), jnp.float32),
            ],
        ),
        out_shape=jax.ShapeDtypeStruct((_B, _H, _D4), jnp.float32),
    )(n_episodes.astype(jnp.int32), episode_ctx, Wp, bp.reshape(1, _D4))

    x = jnp.concatenate([day_embed, hourly.reshape(_B, _H * _D4)], axis=-1)

    out = pl.pallas_call(
        _mlp_body,
        grid=(_NK,),
        in_specs=[
            pl.BlockSpec((_B, _KCHUNK), lambda k: (0, k)),
            pl.BlockSpec((_D, _KCHUNK), lambda k: (0, k)),
            pl.BlockSpec((1, _D), lambda k: (0, 0)),
            pl.BlockSpec((2, _D), lambda k: (0, 0)),
            pl.BlockSpec((1, 2), lambda k: (0, 0)),
        ],
        out_specs=pl.BlockSpec((_B, 2), lambda k: (0, 0)),
        out_shape=jax.ShapeDtypeStruct((_B, 2), jnp.float32),
        scratch_shapes=[pltpu.VMEM((_B, _D), jnp.float32)],
    )(x, W1, b1.reshape(1, _D), W2, b2.reshape(1, 2))

    return out
